# Initial kernel scaffold; baseline (speedup 1.0000x reference)
import jax
import jax.numpy as jnp
from jax.experimental import pallas as pl


def kernel(x, edge_index, batch, W1, b1, W2, b2, W_lin, b_lin):
    raise NotImplementedError("write your pallas kernel here")


# speedup vs baseline: 14.3569x; 14.3569x over previous
import functools

import jax
import jax.numpy as jnp
from jax import lax
from jax.experimental import pallas as pl
from jax.experimental.pallas import tpu as pltpu
from jax.experimental.pallas import tpu_sc as plsc

N = 10000
NPAD = 10240
D = 128
E = 320000
NG = 64
NC = 2
NS = 16
NW = NC * NS
B = 128
K = 80
EPAD = NW * K * B
ROWS_F = NPAD // NS
DUMMY = N + 8
RBLK = 1024
GRID = NPAD // RBLK

_mesh = plsc.VectorSubcoreMesh(core_axis_name="c", subcore_axis_name="s")


def _zero_vec(ref, n):
    zeros16 = jnp.zeros((16,), jnp.float32)

    def body(i, _):
        ref[pl.ds(i * 16, 16)] = zeros16
        return 0

    lax.fori_loop(0, n // 16, body, 0)


def _deg_body(dst_hbm, out_hbm, dstv, vbuf, acc, _):
    cid = lax.axis_index("c")
    sid = lax.axis_index("s")
    wid = sid * NC + cid
    _zero_vec(vbuf, ROWS_F)
    pltpu.sync_copy(vbuf.at[pl.ds(0, ROWS_F)], acc.at[pl.ds(sid * ROWS_F, ROWS_F)])
    pltpu.sync_copy(dst_hbm.at[wid], dstv)
    ones16 = jnp.ones((16,), jnp.float32)
    for i in range(B // 16):
        vbuf[pl.ds(i * 16, 16)] = ones16
    plsc.subcore_barrier()

    def body(j, _):
        pltpu.sync_copy(vbuf.at[pl.ds(0, B)], acc.at[dstv.at[j]], add=True)
        return 0

    lax.fori_loop(0, K, body, 0)
    plsc.subcore_barrier()
    pltpu.sync_copy(acc.at[pl.ds(sid * ROWS_F, ROWS_F)],
                    out_hbm.at[cid, pl.ds(sid * ROWS_F, ROWS_F)])


_deg_call = pl.kernel(
    _deg_body,
    out_type=jax.ShapeDtypeStruct((NC, NPAD), jnp.float32),
    mesh=_mesh,
    scratch_types=[
        pltpu.VMEM((K, B), jnp.int32),
        pltpu.VMEM((ROWS_F,), jnp.float32),
        pltpu.VMEM_SHARED((NPAD,), jnp.float32),
        pltpu.SemaphoreType.DMA,
    ],
)


DH = D // 2


def _msg_body(y_hbm, src_hbm, dst_hbm, out_hbm, srcv, dstv, ybuf, zb, acc, gsem):
    cid = lax.axis_index("c")
    sid = lax.axis_index("s")
    wid = sid * NC + cid
    zeros16 = jnp.zeros((16,), jnp.float32)

    def zrow(i, _):
        for k in range(DH // 16):
            zb[i, pl.ds(k * 16, 16)] = zeros16
        return 0

    lax.fori_loop(0, B, zrow, 0)
    for t in range(ROWS_F // B):
        pltpu.sync_copy(zb, acc.at[pl.ds(sid * ROWS_F + t * B, B)])
    pltpu.sync_copy(src_hbm.at[wid], srcv)
    pltpu.sync_copy(dst_hbm.at[wid], dstv)
    plsc.subcore_barrier()

    def body(j, _):
        pltpu.async_copy(y_hbm.at[srcv.at[j]], ybuf, gsem).wait()
        pltpu.sync_copy(ybuf, acc.at[dstv.at[j]], add=True)
        return 0

    lax.fori_loop(0, K, body, 0)
    plsc.subcore_barrier()
    for t in range(ROWS_F // B):
        pltpu.sync_copy(acc.at[pl.ds(sid * ROWS_F + t * B, B)],
                        out_hbm.at[cid, pl.ds(sid * ROWS_F + t * B, B)])


_msg_call = pl.kernel(
    _msg_body,
    out_type=jax.ShapeDtypeStruct((NC, NPAD, DH), jnp.float32),
    mesh=_mesh,
    compiler_params=pltpu.CompilerParams(use_tc_tiling_on_sc=False),
    scratch_types=[
        pltpu.VMEM((K, B), jnp.int32),
        pltpu.VMEM((K, B), jnp.int32),
        pltpu.VMEM((B, DH), jnp.float32),
        pltpu.VMEM((B, DH), jnp.float32),
        pltpu.VMEM_SHARED((NPAD, DH), jnp.float32),
        pltpu.SemaphoreType.DMA,
    ],
)


def _seg_body(src_hbm, dst_hbm, sd_hbm, out_hbm, srcv, dstv, sdv, svals, vbuf, acc, _):
    cid = lax.axis_index("c")
    sid = lax.axis_index("s")
    wid = sid * NC + cid
    _zero_vec(vbuf, ROWS_F)
    pltpu.sync_copy(vbuf.at[pl.ds(0, ROWS_F)], acc.at[pl.ds(sid * ROWS_F, ROWS_F)])
    pltpu.sync_copy(src_hbm.at[wid], srcv)
    pltpu.sync_copy(dst_hbm.at[wid], dstv)
    pltpu.sync_copy(sd_hbm, sdv)
    plsc.subcore_barrier()

    def body(j, _):
        for i in range(B // 16):
            idx = srcv[j, pl.ds(i * 16, 16)]
            svals[pl.ds(i * 16, 16)] = plsc.load_gather(sdv, [idx])
        pltpu.sync_copy(svals, acc.at[dstv.at[j]], add=True)
        return 0

    lax.fori_loop(0, K, body, 0)
    plsc.subcore_barrier()
    pltpu.sync_copy(acc.at[pl.ds(sid * ROWS_F, ROWS_F)],
                    out_hbm.at[cid, pl.ds(sid * ROWS_F, ROWS_F)])


_seg_call = pl.kernel(
    _seg_body,
    out_type=jax.ShapeDtypeStruct((NC, NPAD), jnp.float32),
    mesh=_mesh,
    compiler_params=pltpu.CompilerParams(needs_layout_passes=False),
    scratch_types=[
        pltpu.VMEM((K, B), jnp.int32),
        pltpu.VMEM((K, B), jnp.int32),
        pltpu.VMEM((NPAD,), jnp.float32),
        pltpu.VMEM((B,), jnp.float32),
        pltpu.VMEM((ROWS_F,), jnp.float32),
        pltpu.VMEM_SHARED((NPAD,), jnp.float32),
        pltpu.SemaphoreType.DMA,
    ],
)


def _pre_body(x_ref, w1_ref, degp_ref, y_ref, dinv_ref):
    deg = degp_ref[0] + degp_ref[1] + 1.0
    dinv = lax.rsqrt(deg)
    xw = jnp.dot(x_ref[...], w1_ref[...], preferred_element_type=jnp.float32)
    y_ref[...] = xw * dinv
    dinv_ref[...] = dinv


_pre_call = pl.pallas_call(
    _pre_body,
    grid=(GRID,),
    in_specs=[
        pl.BlockSpec((RBLK, D), lambda i: (i, 0)),
        pl.BlockSpec((D, D), lambda i: (0, 0)),
        pl.BlockSpec((NC, RBLK, 1), lambda i: (0, i, 0)),
    ],
    out_specs=[
        pl.BlockSpec((RBLK, D), lambda i: (i, 0)),
        pl.BlockSpec((RBLK, 1), lambda i: (i, 0)),
    ],
    out_shape=[
        jax.ShapeDtypeStruct((NPAD, D), jnp.float32),
        jax.ShapeDtypeStruct((NPAD, 1), jnp.float32),
    ],
)


def _mid_body(acc0_ref, acc1_ref, y_ref, dinv_ref, b1_ref, w2_ref, wlin_ref,
              s_ref):
    dinv = dinv_ref[...]
    y = y_ref[...]
    b1 = b1_ref[...]
    w2l = jnp.dot(w2_ref[...], wlin_ref[...], preferred_element_type=jnp.float32)
    a0 = acc0_ref[0] + acc0_ref[1] + y[:, :DH]
    a1 = acc1_ref[0] + acc1_ref[1] + y[:, DH:]
    h0 = jnp.maximum(a0 * dinv + b1[:, :DH], 0.0)
    h1 = jnp.maximum(a1 * dinv + b1[:, DH:], 0.0)
    s = (jnp.dot(h0, w2l[:DH], preferred_element_type=jnp.float32)
         + jnp.dot(h1, w2l[DH:], preferred_element_type=jnp.float32))
    s_ref[...] = s * dinv


_mid_call = pl.pallas_call(
    _mid_body,
    grid=(GRID,),
    in_specs=[
        pl.BlockSpec((NC, RBLK, DH), lambda i: (0, i, 0)),
        pl.BlockSpec((NC, RBLK, DH), lambda i: (0, i, 0)),
        pl.BlockSpec((RBLK, D), lambda i: (i, 0)),
        pl.BlockSpec((RBLK, 1), lambda i: (i, 0)),
        pl.BlockSpec((1, D), lambda i: (0, 0)),
        pl.BlockSpec((D, D), lambda i: (0, 0)),
        pl.BlockSpec((D, 1), lambda i: (0, 0)),
    ],
    out_specs=pl.BlockSpec((RBLK, 1), lambda i: (i, 0)),
    out_shape=jax.ShapeDtypeStruct((NPAD, 1), jnp.float32),
)


def _fin_body(accs_ref, sd_ref, dinv_ref, batch_ref, b2_ref, wlin_ref, blin_ref,
              out_ref):
    z = (accs_ref[0] + accs_ref[1] + sd_ref[...]) * dinv_ref[...]
    c2 = jnp.sum(b2_ref[...] * wlin_ref[...])
    zc = z + c2
    ids = batch_ref[...]
    g = lax.broadcasted_iota(jnp.int32, (NG, 1, 1), 0)
    m = ids[None] == g
    sums = jnp.sum(jnp.where(m, zc[None], 0.0), axis=(1, 2))
    out_ref[...] = sums[:, None] + blin_ref[...]


_fin_call = pl.pallas_call(
    _fin_body,
    out_shape=jax.ShapeDtypeStruct((NG, 1), jnp.float32),
)


def kernel(x, edge_index, batch, W1, b1, W2, b2, W_lin, b_lin):
    src = edge_index[0].astype(jnp.int32)
    dst = edge_index[1].astype(jnp.int32)
    pe = EPAD - E
    src_p = jnp.concatenate([src, jnp.zeros((pe,), jnp.int32)]).reshape(NW, K, B)
    dst_p = jnp.concatenate([dst, jnp.full((pe,), DUMMY, jnp.int32)]).reshape(NW, K, B)
    x_p = jnp.pad(x, ((0, NPAD - N), (0, 0)))
    batch_p = jnp.concatenate(
        [batch.astype(jnp.int32), jnp.full((NPAD - N,), NG, jnp.int32)]
    ).reshape(NPAD // D, D)

    degp = _deg_call(dst_p)
    y, dinv = _pre_call(x_p, W1, degp[:, :, None])
    accp0 = _msg_call(y[:, :DH], src_p, dst_p)
    accp1 = _msg_call(y[:, DH:], src_p, dst_p)
    sdinv = _mid_call(accp0, accp1, y, dinv, b1.reshape(1, D), W2, W_lin)
    accs = _seg_call(src_p, dst_p, sdinv.reshape(NPAD))
    out = _fin_call(
        accs.reshape(NC, NPAD // D, D),
        sdinv.reshape(NPAD // D, D),
        dinv.reshape(NPAD // D, D),
        batch_p,
        jnp.broadcast_to(b2.reshape(D, 1), (D, 1)),
        W_lin,
        b_lin.reshape(1, 1),
    )
    return out


# speedup vs baseline: 15.7574x; 1.0976x over previous
import functools

import jax
import jax.numpy as jnp
from jax import lax
from jax.experimental import pallas as pl
from jax.experimental.pallas import tpu as pltpu
from jax.experimental.pallas import tpu_sc as plsc

N = 10000
NPAD = 10240
D = 128
E = 320000
NG = 64
NC = 2
NS = 16
NW = NC * NS
B = 128
K = 80
EPAD = NW * K * B
ROWS_F = NPAD // NS
DUMMY = N + 8
RBLK = 1024
GRID = NPAD // RBLK

_mesh = plsc.VectorSubcoreMesh(core_axis_name="c", subcore_axis_name="s")


def _zero_vec(ref, n):
    zeros16 = jnp.zeros((16,), jnp.float32)

    def body(i, _):
        ref[pl.ds(i * 16, 16)] = zeros16
        return 0

    lax.fori_loop(0, n // 16, body, 0)


def _deg_body(dst_hbm, out_hbm, dstv, vbuf, acc, _):
    cid = lax.axis_index("c")
    sid = lax.axis_index("s")
    wid = sid * NC + cid
    _zero_vec(vbuf, ROWS_F)
    pltpu.sync_copy(vbuf.at[pl.ds(0, ROWS_F)], acc.at[pl.ds(sid * ROWS_F, ROWS_F)])
    pltpu.sync_copy(dst_hbm.at[wid], dstv)
    ones16 = jnp.ones((16,), jnp.float32)
    for i in range(B // 16):
        vbuf[pl.ds(i * 16, 16)] = ones16
    plsc.subcore_barrier()

    def body(j, _):
        pltpu.sync_copy(vbuf.at[pl.ds(0, B)], acc.at[dstv.at[j]], add=True)
        return 0

    lax.fori_loop(0, K, body, 0)
    plsc.subcore_barrier()
    pltpu.sync_copy(acc.at[pl.ds(sid * ROWS_F, ROWS_F)],
                    out_hbm.at[cid, pl.ds(sid * ROWS_F, ROWS_F)])


_deg_call = pl.kernel(
    _deg_body,
    out_type=jax.ShapeDtypeStruct((NC, NPAD), jnp.float32),
    mesh=_mesh,
    scratch_types=[
        pltpu.VMEM((K, B), jnp.int32),
        pltpu.VMEM((ROWS_F,), jnp.float32),
        pltpu.VMEM_SHARED((NPAD,), jnp.float32),
        pltpu.SemaphoreType.DMA,
    ],
)


DH = D // 2
KT = EPAD // (NS * B)
K0 = 80
NB = 4


def _msg_phase(y_hbm, srcv, dstv, ybuf, acc, gsems, ssems, cs, ce):
    nround = (ce - cs) // NB
    for b in range(NB):
        pltpu.async_copy(y_hbm.at[srcv.at[cs + b]], ybuf.at[b], gsems[b])

    def round_body(t, _):
        j0 = cs + t * NB
        for b in range(NB):
            pltpu.make_async_copy(y_hbm.at[srcv.at[j0 + b]], ybuf.at[b],
                                  gsems[b]).wait()
            pltpu.async_copy(ybuf.at[b], acc.at[dstv.at[j0 + b]], ssems[b],
                             add=True)
        for b in range(NB):
            pltpu.make_async_copy(ybuf.at[b], acc.at[dstv.at[j0 + b]],
                                  ssems[b]).wait()

            @pl.when(t < nround - 1)
            def _():
                pltpu.async_copy(y_hbm.at[srcv.at[j0 + NB + b]], ybuf.at[b],
                                 gsems[b])

        return 0

    lax.fori_loop(0, nround, round_body, 0)


def _msg_zero(zb, acc, sid):
    zeros16 = jnp.zeros((16,), jnp.float32)

    def zrow(i, _):
        for k in range(DH // 16):
            zb[i, pl.ds(k * 16, 16)] = zeros16
        return 0

    lax.fori_loop(0, B, zrow, 0)
    for t in range(ROWS_F // B):
        pltpu.sync_copy(zb, acc.at[pl.ds(sid * ROWS_F + t * B, B)])


def _msg_body(y0_hbm, y1_hbm, src_hbm, dst_hbm, out_hbm, srcv, dstv, ybuf, zb,
              acc, gs0, gs1, gs2, gs3, ss0, ss1, ss2, ss3):
    cid = lax.axis_index("c")
    sid = lax.axis_index("s")
    gsems = (gs0, gs1, gs2, gs3)
    ssems = (ss0, ss1, ss2, ss3)
    _msg_zero(zb, acc, sid)
    pltpu.sync_copy(src_hbm.at[sid], srcv)
    pltpu.sync_copy(dst_hbm.at[sid], dstv)
    plsc.subcore_barrier()
    for h, y_hbm in enumerate((y0_hbm, y1_hbm)):
        @pl.when(cid == 0)
        def _():
            _msg_phase(y_hbm, srcv, dstv, ybuf, acc, gsems, ssems, 0, K0)

        @pl.when(cid == 1)
        def _():
            _msg_phase(y_hbm, srcv, dstv, ybuf, acc, gsems, ssems, K0, KT)

        plsc.subcore_barrier()
        for t in range(ROWS_F // B):
            pltpu.sync_copy(acc.at[pl.ds(sid * ROWS_F + t * B, B)],
                            out_hbm.at[h, cid, pl.ds(sid * ROWS_F + t * B, B)])
        if h == 0:
            plsc.subcore_barrier()
            _msg_zero(zb, acc, sid)
            plsc.subcore_barrier()


_msg_call = pl.kernel(
    _msg_body,
    out_type=jax.ShapeDtypeStruct((2, NC, NPAD, DH), jnp.float32),
    mesh=_mesh,
    compiler_params=pltpu.CompilerParams(use_tc_tiling_on_sc=False),
    scratch_types=[
        pltpu.VMEM((KT, B), jnp.int32),
        pltpu.VMEM((KT, B), jnp.int32),
        pltpu.VMEM((NB, B, DH), jnp.float32),
        pltpu.VMEM((B, DH), jnp.float32),
        pltpu.VMEM_SHARED((NPAD, DH), jnp.float32),
        pltpu.SemaphoreType.DMA,
        pltpu.SemaphoreType.DMA,
        pltpu.SemaphoreType.DMA,
        pltpu.SemaphoreType.DMA,
        pltpu.SemaphoreType.DMA,
        pltpu.SemaphoreType.DMA,
        pltpu.SemaphoreType.DMA,
        pltpu.SemaphoreType.DMA,
    ],
)


def _seg_body(src_hbm, dst_hbm, sd_hbm, out_hbm, srcv, dstv, sdv, svals, vbuf, acc, _):
    cid = lax.axis_index("c")
    sid = lax.axis_index("s")
    wid = sid * NC + cid
    _zero_vec(vbuf, ROWS_F)
    pltpu.sync_copy(vbuf.at[pl.ds(0, ROWS_F)], acc.at[pl.ds(sid * ROWS_F, ROWS_F)])
    pltpu.sync_copy(src_hbm.at[wid], srcv)
    pltpu.sync_copy(dst_hbm.at[wid], dstv)
    pltpu.sync_copy(sd_hbm, sdv)
    plsc.subcore_barrier()

    def body(j, _):
        for i in range(B // 16):
            idx = srcv[j, pl.ds(i * 16, 16)]
            svals[pl.ds(i * 16, 16)] = plsc.load_gather(sdv, [idx])
        pltpu.sync_copy(svals, acc.at[dstv.at[j]], add=True)
        return 0

    lax.fori_loop(0, K, body, 0)
    plsc.subcore_barrier()
    pltpu.sync_copy(acc.at[pl.ds(sid * ROWS_F, ROWS_F)],
                    out_hbm.at[cid, pl.ds(sid * ROWS_F, ROWS_F)])


_seg_call = pl.kernel(
    _seg_body,
    out_type=jax.ShapeDtypeStruct((NC, NPAD), jnp.float32),
    mesh=_mesh,
    compiler_params=pltpu.CompilerParams(needs_layout_passes=False),
    scratch_types=[
        pltpu.VMEM((K, B), jnp.int32),
        pltpu.VMEM((K, B), jnp.int32),
        pltpu.VMEM((NPAD,), jnp.float32),
        pltpu.VMEM((B,), jnp.float32),
        pltpu.VMEM((ROWS_F,), jnp.float32),
        pltpu.VMEM_SHARED((NPAD,), jnp.float32),
        pltpu.SemaphoreType.DMA,
    ],
)


def _pre_body(x_ref, w1_ref, degp_ref, y_ref, dinv_ref):
    deg = degp_ref[0] + degp_ref[1] + 1.0
    dinv = lax.rsqrt(deg)
    xw = jnp.dot(x_ref[...], w1_ref[...], preferred_element_type=jnp.float32)
    y_ref[...] = xw * dinv
    dinv_ref[...] = dinv


_pre_call = pl.pallas_call(
    _pre_body,
    grid=(GRID,),
    in_specs=[
        pl.BlockSpec((RBLK, D), lambda i: (i, 0)),
        pl.BlockSpec((D, D), lambda i: (0, 0)),
        pl.BlockSpec((NC, RBLK, 1), lambda i: (0, i, 0)),
    ],
    out_specs=[
        pl.BlockSpec((RBLK, D), lambda i: (i, 0)),
        pl.BlockSpec((RBLK, 1), lambda i: (i, 0)),
    ],
    out_shape=[
        jax.ShapeDtypeStruct((NPAD, D), jnp.float32),
        jax.ShapeDtypeStruct((NPAD, 1), jnp.float32),
    ],
)


def _mid_body(acc0_ref, acc1_ref, y_ref, dinv_ref, b1_ref, w2_ref, wlin_ref,
              s_ref):
    dinv = dinv_ref[...]
    y = y_ref[...]
    b1 = b1_ref[...]
    w2l = jnp.dot(w2_ref[...], wlin_ref[...], preferred_element_type=jnp.float32)
    a0 = acc0_ref[0] + acc0_ref[1] + y[:, :DH]
    a1 = acc1_ref[0] + acc1_ref[1] + y[:, DH:]
    h0 = jnp.maximum(a0 * dinv + b1[:, :DH], 0.0)
    h1 = jnp.maximum(a1 * dinv + b1[:, DH:], 0.0)
    s = (jnp.dot(h0, w2l[:DH], preferred_element_type=jnp.float32)
         + jnp.dot(h1, w2l[DH:], preferred_element_type=jnp.float32))
    s_ref[...] = s * dinv


_mid_call = pl.pallas_call(
    _mid_body,
    grid=(GRID,),
    in_specs=[
        pl.BlockSpec((NC, RBLK, DH), lambda i: (0, i, 0)),
        pl.BlockSpec((NC, RBLK, DH), lambda i: (0, i, 0)),
        pl.BlockSpec((RBLK, D), lambda i: (i, 0)),
        pl.BlockSpec((RBLK, 1), lambda i: (i, 0)),
        pl.BlockSpec((1, D), lambda i: (0, 0)),
        pl.BlockSpec((D, D), lambda i: (0, 0)),
        pl.BlockSpec((D, 1), lambda i: (0, 0)),
    ],
    out_specs=pl.BlockSpec((RBLK, 1), lambda i: (i, 0)),
    out_shape=jax.ShapeDtypeStruct((NPAD, 1), jnp.float32),
)


def _fin_body(accs_ref, sd_ref, dinv_ref, batch_ref, b2_ref, wlin_ref, blin_ref,
              out_ref):
    z = (accs_ref[0] + accs_ref[1] + sd_ref[...]) * dinv_ref[...]
    c2 = jnp.sum(b2_ref[...] * wlin_ref[...])
    zc = z + c2
    ids = batch_ref[...]
    g = lax.broadcasted_iota(jnp.int32, (NG, 1, 1), 0)
    m = ids[None] == g
    sums = jnp.sum(jnp.where(m, zc[None], 0.0), axis=(1, 2))
    out_ref[...] = sums[:, None] + blin_ref[...]


_fin_call = pl.pallas_call(
    _fin_body,
    out_shape=jax.ShapeDtypeStruct((NG, 1), jnp.float32),
)


def kernel(x, edge_index, batch, W1, b1, W2, b2, W_lin, b_lin):
    src = edge_index[0].astype(jnp.int32)
    dst = edge_index[1].astype(jnp.int32)
    pe = EPAD - E
    src_p = jnp.concatenate([src, jnp.zeros((pe,), jnp.int32)]).reshape(NW, K, B)
    dst_p = jnp.concatenate([dst, jnp.full((pe,), DUMMY, jnp.int32)]).reshape(NW, K, B)
    src_t = src_p.reshape(NS, KT, B)
    dst_t = dst_p.reshape(NS, KT, B)
    x_p = jnp.pad(x, ((0, NPAD - N), (0, 0)))
    batch_p = jnp.concatenate(
        [batch.astype(jnp.int32), jnp.full((NPAD - N,), NG, jnp.int32)]
    ).reshape(NPAD // D, D)

    degp = _deg_call(dst_p)
    y, dinv = _pre_call(x_p, W1, degp[:, :, None])
    accp = _msg_call(y[:, :DH], y[:, DH:], src_t, dst_t)
    sdinv = _mid_call(accp[0], accp[1], y, dinv, b1.reshape(1, D), W2, W_lin)
    accs = _seg_call(src_p, dst_p, sdinv.reshape(NPAD))
    out = _fin_call(
        accs.reshape(NC, NPAD // D, D),
        sdinv.reshape(NPAD // D, D),
        dinv.reshape(NPAD // D, D),
        batch_p,
        jnp.broadcast_to(b2.reshape(D, 1), (D, 1)),
        W_lin,
        b_lin.reshape(1, 1),
    )
    return out


# speedup vs baseline: 18.0984x; 1.1486x over previous
import functools

import jax
import jax.numpy as jnp
from jax import lax
from jax.experimental import pallas as pl
from jax.experimental.pallas import tpu as pltpu
from jax.experimental.pallas import tpu_sc as plsc

N = 10000
NPAD = 10240
D = 128
E = 320000
NG = 64
NC = 2
NS = 16
NW = NC * NS
B = 128
K = 80
EPAD = NW * K * B
ROWS_F = NPAD // NS
DUMMY = N + 8
RBLK = 1024
GRID = NPAD // RBLK

_mesh = plsc.VectorSubcoreMesh(core_axis_name="c", subcore_axis_name="s")


def _zero_vec(ref, n):
    zeros16 = jnp.zeros((16,), jnp.float32)

    def body(i, _):
        ref[pl.ds(i * 16, 16)] = zeros16
        return 0

    lax.fori_loop(0, n // 16, body, 0)


def _deg_body(dst_hbm, out_hbm, dstv, vbuf, acc, _):
    cid = lax.axis_index("c")
    sid = lax.axis_index("s")
    wid = sid * NC + cid
    _zero_vec(vbuf, ROWS_F)
    pltpu.sync_copy(vbuf.at[pl.ds(0, ROWS_F)], acc.at[pl.ds(sid * ROWS_F, ROWS_F)])
    pltpu.sync_copy(dst_hbm.at[wid], dstv)
    ones16 = jnp.ones((16,), jnp.float32)
    for i in range(B // 16):
        vbuf[pl.ds(i * 16, 16)] = ones16
    plsc.subcore_barrier()

    def body(j, _):
        pltpu.sync_copy(vbuf.at[pl.ds(0, B)], acc.at[dstv.at[j]], add=True)
        return 0

    lax.fori_loop(0, K, body, 0)
    plsc.subcore_barrier()
    pltpu.sync_copy(acc.at[pl.ds(sid * ROWS_F, ROWS_F)],
                    out_hbm.at[cid, pl.ds(sid * ROWS_F, ROWS_F)])


_deg_call = pl.kernel(
    _deg_body,
    out_type=jax.ShapeDtypeStruct((NC, NPAD), jnp.float32),
    mesh=_mesh,
    scratch_types=[
        pltpu.VMEM((K, B), jnp.int32),
        pltpu.VMEM((ROWS_F,), jnp.float32),
        pltpu.VMEM_SHARED((NPAD,), jnp.float32),
        pltpu.SemaphoreType.DMA,
    ],
)


DH = D // 2
KT = EPAD // (NS * B)
K0 = 112
NB = 4


def _msg_phase(y_hbm, srcv, dstv, ybuf, acc, gsems, ssems, cs, ce):
    nround = (ce - cs) // NB
    for b in range(NB):
        pltpu.async_copy(y_hbm.at[srcv.at[cs + b]], ybuf.at[b], gsems[b])

    def round_body(t, _):
        j0 = cs + t * NB
        for b in range(NB):
            pltpu.make_async_copy(y_hbm.at[srcv.at[j0 + b]], ybuf.at[b],
                                  gsems[b]).wait()
            pltpu.async_copy(ybuf.at[b], acc.at[dstv.at[j0 + b]], ssems[b],
                             add=True)
        for b in range(NB):
            pltpu.make_async_copy(ybuf.at[b], acc.at[dstv.at[j0 + b]],
                                  ssems[b]).wait()

            @pl.when(t < nround - 1)
            def _():
                pltpu.async_copy(y_hbm.at[srcv.at[j0 + NB + b]], ybuf.at[b],
                                 gsems[b])

        return 0

    lax.fori_loop(0, nround, round_body, 0)


def _msg_zero(zb, acc, sid):
    zeros16 = jnp.zeros((16,), jnp.float32)

    def zrow(i, _):
        for k in range(DH // 16):
            zb[i, pl.ds(k * 16, 16)] = zeros16
        return 0

    lax.fori_loop(0, B, zrow, 0)
    for t in range(ROWS_F // B):
        pltpu.sync_copy(zb, acc.at[pl.ds(sid * ROWS_F + t * B, B)])


def _msg_body(y0_hbm, y1_hbm, src_hbm, dst_hbm, out_hbm, srcv, dstv, ybuf, zb,
              acc, gs0, gs1, gs2, gs3, ss0, ss1, ss2, ss3):
    cid = lax.axis_index("c")
    sid = lax.axis_index("s")
    gsems = (gs0, gs1, gs2, gs3)
    ssems = (ss0, ss1, ss2, ss3)
    _msg_zero(zb, acc, sid)
    pltpu.sync_copy(src_hbm.at[sid], srcv)
    pltpu.sync_copy(dst_hbm.at[sid], dstv)
    plsc.subcore_barrier()
    for h, y_hbm in enumerate((y0_hbm, y1_hbm)):
        @pl.when(cid == 0)
        def _():
            _msg_phase(y_hbm, srcv, dstv, ybuf, acc, gsems, ssems, 0, K0)

        @pl.when(cid == 1)
        def _():
            _msg_phase(y_hbm, srcv, dstv, ybuf, acc, gsems, ssems, K0, KT)

        plsc.subcore_barrier()
        for t in range(ROWS_F // B):
            pltpu.sync_copy(acc.at[pl.ds(sid * ROWS_F + t * B, B)],
                            out_hbm.at[h, cid, pl.ds(sid * ROWS_F + t * B, B)])
        if h == 0:
            plsc.subcore_barrier()
            _msg_zero(zb, acc, sid)
            plsc.subcore_barrier()


_msg_call = pl.kernel(
    _msg_body,
    out_type=jax.ShapeDtypeStruct((2, NC, NPAD, DH), jnp.float32),
    mesh=_mesh,
    compiler_params=pltpu.CompilerParams(use_tc_tiling_on_sc=False),
    scratch_types=[
        pltpu.VMEM((KT, B), jnp.int32),
        pltpu.VMEM((KT, B), jnp.int32),
        pltpu.VMEM((NB, B, DH), jnp.float32),
        pltpu.VMEM((B, DH), jnp.float32),
        pltpu.VMEM_SHARED((NPAD, DH), jnp.float32),
        pltpu.SemaphoreType.DMA,
        pltpu.SemaphoreType.DMA,
        pltpu.SemaphoreType.DMA,
        pltpu.SemaphoreType.DMA,
        pltpu.SemaphoreType.DMA,
        pltpu.SemaphoreType.DMA,
        pltpu.SemaphoreType.DMA,
        pltpu.SemaphoreType.DMA,
    ],
)


def _seg_body(src_hbm, dst_hbm, sd_hbm, out_hbm, srcv, dstv, sdv, svals, vbuf, acc, _):
    cid = lax.axis_index("c")
    sid = lax.axis_index("s")
    wid = sid * NC + cid
    _zero_vec(vbuf, ROWS_F)
    pltpu.sync_copy(vbuf.at[pl.ds(0, ROWS_F)], acc.at[pl.ds(sid * ROWS_F, ROWS_F)])
    pltpu.sync_copy(src_hbm.at[wid], srcv)
    pltpu.sync_copy(dst_hbm.at[wid], dstv)
    pltpu.sync_copy(sd_hbm, sdv)
    plsc.subcore_barrier()

    def body(j, _):
        for i in range(B // 16):
            idx = srcv[j, pl.ds(i * 16, 16)]
            svals[pl.ds(i * 16, 16)] = plsc.load_gather(sdv, [idx])
        pltpu.sync_copy(svals, acc.at[dstv.at[j]], add=True)
        return 0

    lax.fori_loop(0, K, body, 0)
    plsc.subcore_barrier()
    pltpu.sync_copy(acc.at[pl.ds(sid * ROWS_F, ROWS_F)],
                    out_hbm.at[cid, pl.ds(sid * ROWS_F, ROWS_F)])


_seg_call = pl.kernel(
    _seg_body,
    out_type=jax.ShapeDtypeStruct((NC, NPAD), jnp.float32),
    mesh=_mesh,
    compiler_params=pltpu.CompilerParams(needs_layout_passes=False),
    scratch_types=[
        pltpu.VMEM((K, B), jnp.int32),
        pltpu.VMEM((K, B), jnp.int32),
        pltpu.VMEM((NPAD,), jnp.float32),
        pltpu.VMEM((B,), jnp.float32),
        pltpu.VMEM((ROWS_F,), jnp.float32),
        pltpu.VMEM_SHARED((NPAD,), jnp.float32),
        pltpu.SemaphoreType.DMA,
    ],
)


def _pre_body(x_ref, w1_ref, degp_ref, y_ref, dinv_ref):
    deg = degp_ref[0] + degp_ref[1] + 1.0
    dinv = lax.rsqrt(deg)
    xw = jnp.dot(x_ref[...], w1_ref[...], preferred_element_type=jnp.float32)
    y_ref[...] = xw * dinv
    dinv_ref[...] = dinv


_pre_call = pl.pallas_call(
    _pre_body,
    grid=(GRID,),
    in_specs=[
        pl.BlockSpec((RBLK, D), lambda i: (i, 0)),
        pl.BlockSpec((D, D), lambda i: (0, 0)),
        pl.BlockSpec((NC, RBLK, 1), lambda i: (0, i, 0)),
    ],
    out_specs=[
        pl.BlockSpec((RBLK, D), lambda i: (i, 0)),
        pl.BlockSpec((RBLK, 1), lambda i: (i, 0)),
    ],
    out_shape=[
        jax.ShapeDtypeStruct((NPAD, D), jnp.float32),
        jax.ShapeDtypeStruct((NPAD, 1), jnp.float32),
    ],
)


def _mid_body(acc0_ref, acc1_ref, y_ref, dinv_ref, b1_ref, w2_ref, wlin_ref,
              s_ref):
    dinv = dinv_ref[...]
    y = y_ref[...]
    b1 = b1_ref[...]
    w2l = jnp.dot(w2_ref[...], wlin_ref[...], preferred_element_type=jnp.float32)
    a0 = acc0_ref[0] + acc0_ref[1] + y[:, :DH]
    a1 = acc1_ref[0] + acc1_ref[1] + y[:, DH:]
    h0 = jnp.maximum(a0 * dinv + b1[:, :DH], 0.0)
    h1 = jnp.maximum(a1 * dinv + b1[:, DH:], 0.0)
    s = (jnp.dot(h0, w2l[:DH], preferred_element_type=jnp.float32)
         + jnp.dot(h1, w2l[DH:], preferred_element_type=jnp.float32))
    s_ref[...] = s * dinv


_mid_call = pl.pallas_call(
    _mid_body,
    grid=(GRID,),
    in_specs=[
        pl.BlockSpec((NC, RBLK, DH), lambda i: (0, i, 0)),
        pl.BlockSpec((NC, RBLK, DH), lambda i: (0, i, 0)),
        pl.BlockSpec((RBLK, D), lambda i: (i, 0)),
        pl.BlockSpec((RBLK, 1), lambda i: (i, 0)),
        pl.BlockSpec((1, D), lambda i: (0, 0)),
        pl.BlockSpec((D, D), lambda i: (0, 0)),
        pl.BlockSpec((D, 1), lambda i: (0, 0)),
    ],
    out_specs=pl.BlockSpec((RBLK, 1), lambda i: (i, 0)),
    out_shape=jax.ShapeDtypeStruct((NPAD, 1), jnp.float32),
)


def _fin_body(accs_ref, sd_ref, dinv_ref, batch_ref, b2_ref, wlin_ref, blin_ref,
              out_ref):
    z = (accs_ref[0] + accs_ref[1] + sd_ref[...]) * dinv_ref[...]
    c2 = jnp.sum(b2_ref[...] * wlin_ref[...])
    zc = z + c2
    ids = batch_ref[...]
    g = lax.broadcasted_iota(jnp.int32, (NG, 1, 1), 0)
    m = ids[None] == g
    sums = jnp.sum(jnp.where(m, zc[None], 0.0), axis=(1, 2))
    out_ref[...] = sums[:, None] + blin_ref[...]


_fin_call = pl.pallas_call(
    _fin_body,
    out_shape=jax.ShapeDtypeStruct((NG, 1), jnp.float32),
)


def kernel(x, edge_index, batch, W1, b1, W2, b2, W_lin, b_lin):
    src = edge_index[0].astype(jnp.int32)
    dst = edge_index[1].astype(jnp.int32)
    pe = EPAD - E
    src_p = jnp.concatenate([src, jnp.zeros((pe,), jnp.int32)]).reshape(NW, K, B)
    dst_p = jnp.concatenate([dst, jnp.full((pe,), DUMMY, jnp.int32)]).reshape(NW, K, B)
    src_t = src_p.reshape(NS, KT, B)
    dst_t = dst_p.reshape(NS, KT, B)
    x_p = jnp.pad(x, ((0, NPAD - N), (0, 0)))
    batch_p = jnp.concatenate(
        [batch.astype(jnp.int32), jnp.full((NPAD - N,), NG, jnp.int32)]
    ).reshape(NPAD // D, D)

    degp = _deg_call(dst_p)
    y, dinv = _pre_call(x_p, W1, degp[:, :, None])
    accp = _msg_call(y[:, :DH], y[:, DH:], src_t, dst_t)
    sdinv = _mid_call(accp[0], accp[1], y, dinv, b1.reshape(1, D), W2, W_lin)
    accs = _seg_call(src_p, dst_p, sdinv.reshape(NPAD))
    out = _fin_call(
        accs.reshape(NC, NPAD // D, D),
        sdinv.reshape(NPAD // D, D),
        dinv.reshape(NPAD // D, D),
        batch_p,
        jnp.broadcast_to(b2.reshape(D, 1), (D, 1)),
        W_lin,
        b_lin.reshape(1, 1),
    )
    return out


# speedup vs baseline: 18.4785x; 1.0210x over previous
import functools

import jax
import jax.numpy as jnp
from jax import lax
from jax.experimental import pallas as pl
from jax.experimental.pallas import tpu as pltpu
from jax.experimental.pallas import tpu_sc as plsc

N = 10000
NPAD = 10240
D = 128
E = 320000
NG = 64
NC = 2
NS = 16
NW = NC * NS
B = 128
K = 80
EPAD = NW * K * B
ROWS_F = NPAD // NS
DUMMY = N + 8
RBLK = 1024
GRID = NPAD // RBLK

_mesh = plsc.VectorSubcoreMesh(core_axis_name="c", subcore_axis_name="s")


def _zero_vec(ref, n):
    zeros16 = jnp.zeros((16,), jnp.float32)

    def body(i, _):
        ref[pl.ds(i * 16, 16)] = zeros16
        return 0

    lax.fori_loop(0, n // 16, body, 0)


def _deg_body(dst_hbm, out_hbm, dstv, vbuf, acc, _):
    cid = lax.axis_index("c")
    sid = lax.axis_index("s")
    wid = sid * NC + cid
    _zero_vec(vbuf, ROWS_F)
    pltpu.sync_copy(vbuf.at[pl.ds(0, ROWS_F)], acc.at[pl.ds(sid * ROWS_F, ROWS_F)])
    pltpu.sync_copy(dst_hbm.at[wid], dstv)
    ones16 = jnp.ones((16,), jnp.float32)
    for i in range(B // 16):
        vbuf[pl.ds(i * 16, 16)] = ones16
    plsc.subcore_barrier()

    def body(j, _):
        pltpu.sync_copy(vbuf.at[pl.ds(0, B)], acc.at[dstv.at[j]], add=True)
        return 0

    lax.fori_loop(0, K, body, 0)
    plsc.subcore_barrier()
    pltpu.sync_copy(acc.at[pl.ds(sid * ROWS_F, ROWS_F)],
                    out_hbm.at[cid, pl.ds(sid * ROWS_F, ROWS_F)])


_deg_call = pl.kernel(
    _deg_body,
    out_type=jax.ShapeDtypeStruct((NC, NPAD), jnp.float32),
    mesh=_mesh,
    scratch_types=[
        pltpu.VMEM((K, B), jnp.int32),
        pltpu.VMEM((ROWS_F,), jnp.float32),
        pltpu.VMEM_SHARED((NPAD,), jnp.float32),
        pltpu.SemaphoreType.DMA,
    ],
)


DH = D // 2
KT = EPAD // (NS * B)
K0 = 128
NB = 4


def _msg_phase(y_hbm, srcv, dstv, ybuf, acc, gsems, ssems, cs, ce):
    nround = (ce - cs) // NB
    for b in range(NB):
        pltpu.async_copy(y_hbm.at[srcv.at[cs + b]], ybuf.at[b], gsems[b])

    def round_body(t, _):
        j0 = cs + t * NB
        for b in range(NB):
            pltpu.make_async_copy(y_hbm.at[srcv.at[j0 + b]], ybuf.at[b],
                                  gsems[b]).wait()
            pltpu.async_copy(ybuf.at[b], acc.at[dstv.at[j0 + b]], ssems[b],
                             add=True)
        for b in range(NB):
            pltpu.make_async_copy(ybuf.at[b], acc.at[dstv.at[j0 + b]],
                                  ssems[b]).wait()

            @pl.when(t < nround - 1)
            def _():
                pltpu.async_copy(y_hbm.at[srcv.at[j0 + NB + b]], ybuf.at[b],
                                 gsems[b])

        return 0

    lax.fori_loop(0, nround, round_body, 0)


def _msg_zero(zb, acc, sid):
    zeros16 = jnp.zeros((16,), jnp.float32)

    def zrow(i, _):
        for k in range(DH // 16):
            zb[i, pl.ds(k * 16, 16)] = zeros16
        return 0

    lax.fori_loop(0, B, zrow, 0)
    for t in range(ROWS_F // B):
        pltpu.sync_copy(zb, acc.at[pl.ds(sid * ROWS_F + t * B, B)])


def _msg_body(y0_hbm, y1_hbm, src_hbm, dst_hbm, out_hbm, srcv, dstv, ybuf, zb,
              acc, gs0, gs1, gs2, gs3, ss0, ss1, ss2, ss3):
    cid = lax.axis_index("c")
    sid = lax.axis_index("s")
    gsems = (gs0, gs1, gs2, gs3)
    ssems = (ss0, ss1, ss2, ss3)
    _msg_zero(zb, acc, sid)
    pltpu.sync_copy(src_hbm.at[sid], srcv)
    pltpu.sync_copy(dst_hbm.at[sid], dstv)
    plsc.subcore_barrier()
    for h, y_hbm in enumerate((y0_hbm, y1_hbm)):
        @pl.when(cid == 0)
        def _():
            _msg_phase(y_hbm, srcv, dstv, ybuf, acc, gsems, ssems, 0, K0)

        @pl.when(cid == 1)
        def _():
            _msg_phase(y_hbm, srcv, dstv, ybuf, acc, gsems, ssems, K0, KT)

        plsc.subcore_barrier()
        for t in range(ROWS_F // B):
            pltpu.sync_copy(acc.at[pl.ds(sid * ROWS_F + t * B, B)],
                            out_hbm.at[h, cid, pl.ds(sid * ROWS_F + t * B, B)])
        if h == 0:
            plsc.subcore_barrier()
            _msg_zero(zb, acc, sid)
            plsc.subcore_barrier()


_msg_call = pl.kernel(
    _msg_body,
    out_type=jax.ShapeDtypeStruct((2, NC, NPAD, DH), jnp.float32),
    mesh=_mesh,
    compiler_params=pltpu.CompilerParams(use_tc_tiling_on_sc=False),
    scratch_types=[
        pltpu.VMEM((KT, B), jnp.int32),
        pltpu.VMEM((KT, B), jnp.int32),
        pltpu.VMEM((NB, B, DH), jnp.float32),
        pltpu.VMEM((B, DH), jnp.float32),
        pltpu.VMEM_SHARED((NPAD, DH), jnp.float32),
        pltpu.SemaphoreType.DMA,
        pltpu.SemaphoreType.DMA,
        pltpu.SemaphoreType.DMA,
        pltpu.SemaphoreType.DMA,
        pltpu.SemaphoreType.DMA,
        pltpu.SemaphoreType.DMA,
        pltpu.SemaphoreType.DMA,
        pltpu.SemaphoreType.DMA,
    ],
)


def _seg_body(src_hbm, dst_hbm, sd_hbm, out_hbm, srcv, dstv, sdv, svals, vbuf, acc, _):
    cid = lax.axis_index("c")
    sid = lax.axis_index("s")
    wid = sid * NC + cid
    _zero_vec(vbuf, ROWS_F)
    pltpu.sync_copy(vbuf.at[pl.ds(0, ROWS_F)], acc.at[pl.ds(sid * ROWS_F, ROWS_F)])
    pltpu.sync_copy(src_hbm.at[wid], srcv)
    pltpu.sync_copy(dst_hbm.at[wid], dstv)
    pltpu.sync_copy(sd_hbm, sdv)
    plsc.subcore_barrier()

    def body(j, _):
        for i in range(B // 16):
            idx = srcv[j, pl.ds(i * 16, 16)]
            svals[pl.ds(i * 16, 16)] = plsc.load_gather(sdv, [idx])
        pltpu.sync_copy(svals, acc.at[dstv.at[j]], add=True)
        return 0

    lax.fori_loop(0, K, body, 0)
    plsc.subcore_barrier()
    pltpu.sync_copy(acc.at[pl.ds(sid * ROWS_F, ROWS_F)],
                    out_hbm.at[cid, pl.ds(sid * ROWS_F, ROWS_F)])


_seg_call = pl.kernel(
    _seg_body,
    out_type=jax.ShapeDtypeStruct((NC, NPAD), jnp.float32),
    mesh=_mesh,
    compiler_params=pltpu.CompilerParams(needs_layout_passes=False),
    scratch_types=[
        pltpu.VMEM((K, B), jnp.int32),
        pltpu.VMEM((K, B), jnp.int32),
        pltpu.VMEM((NPAD,), jnp.float32),
        pltpu.VMEM((B,), jnp.float32),
        pltpu.VMEM((ROWS_F,), jnp.float32),
        pltpu.VMEM_SHARED((NPAD,), jnp.float32),
        pltpu.SemaphoreType.DMA,
    ],
)


def _pre_body(x_ref, w1_ref, degp_ref, y_ref, dinv_ref):
    deg = degp_ref[0] + degp_ref[1] + 1.0
    dinv = lax.rsqrt(deg)
    xw = jnp.dot(x_ref[...], w1_ref[...], preferred_element_type=jnp.float32)
    y_ref[...] = xw * dinv
    dinv_ref[...] = dinv


_pre_call = pl.pallas_call(
    _pre_body,
    grid=(GRID,),
    in_specs=[
        pl.BlockSpec((RBLK, D), lambda i: (i, 0)),
        pl.BlockSpec((D, D), lambda i: (0, 0)),
        pl.BlockSpec((NC, RBLK, 1), lambda i: (0, i, 0)),
    ],
    out_specs=[
        pl.BlockSpec((RBLK, D), lambda i: (i, 0)),
        pl.BlockSpec((RBLK, 1), lambda i: (i, 0)),
    ],
    out_shape=[
        jax.ShapeDtypeStruct((NPAD, D), jnp.float32),
        jax.ShapeDtypeStruct((NPAD, 1), jnp.float32),
    ],
)


def _mid_body(acc0_ref, acc1_ref, y_ref, dinv_ref, b1_ref, w2_ref, wlin_ref,
              s_ref):
    dinv = dinv_ref[...]
    y = y_ref[...]
    b1 = b1_ref[...]
    w2l = jnp.dot(w2_ref[...], wlin_ref[...], preferred_element_type=jnp.float32)
    a0 = acc0_ref[0] + acc0_ref[1] + y[:, :DH]
    a1 = acc1_ref[0] + acc1_ref[1] + y[:, DH:]
    h0 = jnp.maximum(a0 * dinv + b1[:, :DH], 0.0)
    h1 = jnp.maximum(a1 * dinv + b1[:, DH:], 0.0)
    s = (jnp.dot(h0, w2l[:DH], preferred_element_type=jnp.float32)
         + jnp.dot(h1, w2l[DH:], preferred_element_type=jnp.float32))
    s_ref[...] = s * dinv


_mid_call = pl.pallas_call(
    _mid_body,
    grid=(GRID,),
    in_specs=[
        pl.BlockSpec((NC, RBLK, DH), lambda i: (0, i, 0)),
        pl.BlockSpec((NC, RBLK, DH), lambda i: (0, i, 0)),
        pl.BlockSpec((RBLK, D), lambda i: (i, 0)),
        pl.BlockSpec((RBLK, 1), lambda i: (i, 0)),
        pl.BlockSpec((1, D), lambda i: (0, 0)),
        pl.BlockSpec((D, D), lambda i: (0, 0)),
        pl.BlockSpec((D, 1), lambda i: (0, 0)),
    ],
    out_specs=pl.BlockSpec((RBLK, 1), lambda i: (i, 0)),
    out_shape=jax.ShapeDtypeStruct((NPAD, 1), jnp.float32),
)


def _fin_body(accs_ref, sd_ref, dinv_ref, batch_ref, b2_ref, wlin_ref, blin_ref,
              out_ref):
    z = (accs_ref[0] + accs_ref[1] + sd_ref[...]) * dinv_ref[...]
    c2 = jnp.sum(b2_ref[...] * wlin_ref[...])
    zc = z + c2
    ids = batch_ref[...]
    g = lax.broadcasted_iota(jnp.int32, (NG, 1, 1), 0)
    m = ids[None] == g
    sums = jnp.sum(jnp.where(m, zc[None], 0.0), axis=(1, 2))
    out_ref[...] = sums[:, None] + blin_ref[...]


_fin_call = pl.pallas_call(
    _fin_body,
    out_shape=jax.ShapeDtypeStruct((NG, 1), jnp.float32),
)


def kernel(x, edge_index, batch, W1, b1, W2, b2, W_lin, b_lin):
    src = edge_index[0].astype(jnp.int32)
    dst = edge_index[1].astype(jnp.int32)
    pe = EPAD - E
    src_p = jnp.concatenate([src, jnp.zeros((pe,), jnp.int32)]).reshape(NW, K, B)
    dst_p = jnp.concatenate([dst, jnp.full((pe,), DUMMY, jnp.int32)]).reshape(NW, K, B)
    src_t = src_p.reshape(NS, KT, B)
    dst_t = dst_p.reshape(NS, KT, B)
    x_p = jnp.pad(x, ((0, NPAD - N), (0, 0)))
    batch_p = jnp.concatenate(
        [batch.astype(jnp.int32), jnp.full((NPAD - N,), NG, jnp.int32)]
    ).reshape(NPAD // D, D)

    degp = _deg_call(dst_p)
    y, dinv = _pre_call(x_p, W1, degp[:, :, None])
    accp = _msg_call(y[:, :DH], y[:, DH:], src_t, dst_t)
    sdinv = _mid_call(accp[0], accp[1], y, dinv, b1.reshape(1, D), W2, W_lin)
    accs = _seg_call(src_p, dst_p, sdinv.reshape(NPAD))
    out = _fin_call(
        accs.reshape(NC, NPAD // D, D),
        sdinv.reshape(NPAD // D, D),
        dinv.reshape(NPAD // D, D),
        batch_p,
        jnp.broadcast_to(b2.reshape(D, 1), (D, 1)),
        W_lin,
        b_lin.reshape(1, 1),
    )
    return out


# speedup vs baseline: 37.4570x; 2.0271x over previous
import functools

import jax
import jax.numpy as jnp
from jax import lax
from jax.experimental import pallas as pl
from jax.experimental.pallas import tpu as pltpu
from jax.experimental.pallas import tpu_sc as plsc

N = 10000
NPAD = 10240
D = 128
E = 320000
NG = 64
NC = 2
NS = 16
NW = NC * NS
B = 128
K = 80
EPAD = NW * K * B
ROWS_F = NPAD // NS
DUMMY = N + 8
RBLK = 1024
GRID = NPAD // RBLK

_mesh = plsc.VectorSubcoreMesh(core_axis_name="c", subcore_axis_name="s")


def _zero_vec(ref, n):
    zeros16 = jnp.zeros((16,), jnp.float32)

    def body(i, _):
        ref[pl.ds(i * 16, 16)] = zeros16
        return 0

    lax.fori_loop(0, n // 16, body, 0)


def _deg_body(dst_hbm, out_hbm, dstv, vbuf, acc, _):
    cid = lax.axis_index("c")
    sid = lax.axis_index("s")
    wid = sid * NC + cid
    _zero_vec(vbuf, ROWS_F)
    pltpu.sync_copy(vbuf.at[pl.ds(0, ROWS_F)], acc.at[pl.ds(sid * ROWS_F, ROWS_F)])
    pltpu.sync_copy(dst_hbm.at[wid], dstv)
    ones16 = jnp.ones((16,), jnp.float32)
    for i in range(B // 16):
        vbuf[pl.ds(i * 16, 16)] = ones16
    plsc.subcore_barrier()

    def body(j, _):
        pltpu.sync_copy(vbuf.at[pl.ds(0, B)], acc.at[dstv.at[j]], add=True)
        return 0

    lax.fori_loop(0, K, body, 0)
    plsc.subcore_barrier()
    pltpu.sync_copy(acc.at[pl.ds(sid * ROWS_F, ROWS_F)],
                    out_hbm.at[cid, pl.ds(sid * ROWS_F, ROWS_F)])


_deg_call = pl.kernel(
    _deg_body,
    out_type=jax.ShapeDtypeStruct((NC, NPAD), jnp.float32),
    mesh=_mesh,
    scratch_types=[
        pltpu.VMEM((K, B), jnp.int32),
        pltpu.VMEM((ROWS_F,), jnp.float32),
        pltpu.VMEM_SHARED((NPAD,), jnp.float32),
        pltpu.SemaphoreType.DMA,
    ],
)


DH = D // 2
KT = EPAD // (NS * B)
K0 = 80
NB = 4


def _msg_phase(y_hbm, srcv, dstv, ybuf, acc, gsems, ssems, cs, ce):
    nround = (ce - cs) // NB
    for b in range(NB):
        pltpu.async_copy(y_hbm.at[srcv.at[cs + b]], ybuf.at[b], gsems[b])

    def round_body(t, _):
        j0 = cs + t * NB
        for b in range(NB):
            pltpu.make_async_copy(y_hbm.at[srcv.at[j0 + b]], ybuf.at[b],
                                  gsems[b]).wait()
            pltpu.async_copy(ybuf.at[b], acc.at[dstv.at[j0 + b]], ssems[b],
                             add=True)
        for b in range(NB):
            pltpu.make_async_copy(ybuf.at[b], acc.at[dstv.at[j0 + b]],
                                  ssems[b]).wait()

            @pl.when(t < nround - 1)
            def _():
                pltpu.async_copy(y_hbm.at[srcv.at[j0 + NB + b]], ybuf.at[b],
                                 gsems[b])

        return 0

    lax.fori_loop(0, nround, round_body, 0)


def _msg_zero(zb, acc, sid):
    zeros16 = jnp.zeros((16,), jnp.float32)

    def zrow(i, _):
        for k in range(DH // 16):
            zb[i, pl.ds(k * 16, 16)] = zeros16
        return 0

    lax.fori_loop(0, B, zrow, 0)
    for t in range(ROWS_F // B):
        pltpu.sync_copy(zb, acc.at[pl.ds(sid * ROWS_F + t * B, B)])


def _msg_body(y0_hbm, y1_hbm, src_hbm, dst_hbm, out_hbm, srcv, dstv, ybuf, zb,
              acc, gs0, gs1, gs2, gs3, ss0, ss1, ss2, ss3):
    cid = lax.axis_index("c")
    sid = lax.axis_index("s")
    gsems = (gs0, gs1, gs2, gs3)
    ssems = (ss0, ss1, ss2, ss3)
    _msg_zero(zb, acc, sid)
    pltpu.sync_copy(src_hbm.at[sid], srcv)
    pltpu.sync_copy(dst_hbm.at[sid], dstv)
    plsc.subcore_barrier()
    for h, y_hbm in enumerate((y0_hbm, y1_hbm)):
        @pl.when(cid == 0)
        def _():
            _msg_phase(y_hbm, srcv, dstv, ybuf, acc, gsems, ssems, 0, K0)

        @pl.when(cid == 1)
        def _():
            _msg_phase(y_hbm, srcv, dstv, ybuf, acc, gsems, ssems, K0, KT)

        plsc.subcore_barrier()
        for t in range(ROWS_F // B):
            pltpu.sync_copy(acc.at[pl.ds(sid * ROWS_F + t * B, B)],
                            out_hbm.at[h, cid, pl.ds(sid * ROWS_F + t * B, B)])
        if h == 0:
            plsc.subcore_barrier()
            _msg_zero(zb, acc, sid)
            plsc.subcore_barrier()


_msg_call = pl.kernel(
    _msg_body,
    out_type=jax.ShapeDtypeStruct((2, NC, NPAD, DH), jnp.float32),
    mesh=_mesh,
    compiler_params=pltpu.CompilerParams(use_tc_tiling_on_sc=False),
    scratch_types=[
        pltpu.VMEM((KT, B), jnp.int32),
        pltpu.VMEM((KT, B), jnp.int32),
        pltpu.VMEM((NB, B, DH), jnp.float32),
        pltpu.VMEM((B, DH), jnp.float32),
        pltpu.VMEM_SHARED((NPAD, DH), jnp.float32),
        pltpu.SemaphoreType.DMA,
        pltpu.SemaphoreType.DMA,
        pltpu.SemaphoreType.DMA,
        pltpu.SemaphoreType.DMA,
        pltpu.SemaphoreType.DMA,
        pltpu.SemaphoreType.DMA,
        pltpu.SemaphoreType.DMA,
        pltpu.SemaphoreType.DMA,
    ],
)


def _seg_body(src_hbm, dst_hbm, sd_hbm, out_hbm, srcv, dstv, sdv, svals, vbuf, acc, _):
    cid = lax.axis_index("c")
    sid = lax.axis_index("s")
    wid = sid * NC + cid
    _zero_vec(vbuf, ROWS_F)
    pltpu.sync_copy(vbuf.at[pl.ds(0, ROWS_F)], acc.at[pl.ds(sid * ROWS_F, ROWS_F)])
    pltpu.sync_copy(src_hbm.at[wid], srcv)
    pltpu.sync_copy(dst_hbm.at[wid], dstv)
    pltpu.sync_copy(sd_hbm, sdv)
    plsc.subcore_barrier()

    def body(j, _):
        for i in range(B // 16):
            idx = srcv[j, pl.ds(i * 16, 16)]
            svals[pl.ds(i * 16, 16)] = plsc.load_gather(sdv, [idx])
        pltpu.sync_copy(svals, acc.at[dstv.at[j]], add=True)
        return 0

    lax.fori_loop(0, K, body, 0)
    plsc.subcore_barrier()
    pltpu.sync_copy(acc.at[pl.ds(sid * ROWS_F, ROWS_F)],
                    out_hbm.at[cid, pl.ds(sid * ROWS_F, ROWS_F)])


_seg_call = pl.kernel(
    _seg_body,
    out_type=jax.ShapeDtypeStruct((NC, NPAD), jnp.float32),
    mesh=_mesh,
    compiler_params=pltpu.CompilerParams(needs_layout_passes=False),
    scratch_types=[
        pltpu.VMEM((K, B), jnp.int32),
        pltpu.VMEM((K, B), jnp.int32),
        pltpu.VMEM((NPAD,), jnp.float32),
        pltpu.VMEM((B,), jnp.float32),
        pltpu.VMEM((ROWS_F,), jnp.float32),
        pltpu.VMEM_SHARED((NPAD,), jnp.float32),
        pltpu.SemaphoreType.DMA,
    ],
)


def _pre_body(x_ref, w1_ref, degp_ref, y_ref, dinv_ref):
    deg = degp_ref[0] + degp_ref[1] + 1.0
    dinv = lax.rsqrt(deg)
    xw = jnp.dot(x_ref[...], w1_ref[...], preferred_element_type=jnp.float32)
    y_ref[...] = xw * dinv
    dinv_ref[...] = dinv


_pre_call = pl.pallas_call(
    _pre_body,
    grid=(GRID,),
    in_specs=[
        pl.BlockSpec((RBLK, D), lambda i: (i, 0)),
        pl.BlockSpec((D, D), lambda i: (0, 0)),
        pl.BlockSpec((NC, RBLK, 1), lambda i: (0, i, 0)),
    ],
    out_specs=[
        pl.BlockSpec((RBLK, D), lambda i: (i, 0)),
        pl.BlockSpec((RBLK, 1), lambda i: (i, 0)),
    ],
    out_shape=[
        jax.ShapeDtypeStruct((NPAD, D), jnp.float32),
        jax.ShapeDtypeStruct((NPAD, 1), jnp.float32),
    ],
)


def _mid_body(acc0_ref, acc1_ref, y_ref, dinv_ref, b1_ref, w2_ref, wlin_ref,
              s_ref):
    dinv = dinv_ref[...]
    y = y_ref[...]
    b1 = b1_ref[...]
    w2l = jnp.dot(w2_ref[...], wlin_ref[...], preferred_element_type=jnp.float32)
    a0 = acc0_ref[0] + acc0_ref[1] + y[:, :DH]
    a1 = acc1_ref[0] + acc1_ref[1] + y[:, DH:]
    h0 = jnp.maximum(a0 * dinv + b1[:, :DH], 0.0)
    h1 = jnp.maximum(a1 * dinv + b1[:, DH:], 0.0)
    s = (jnp.dot(h0, w2l[:DH], preferred_element_type=jnp.float32)
         + jnp.dot(h1, w2l[DH:], preferred_element_type=jnp.float32))
    s_ref[...] = s * dinv


_mid_call = pl.pallas_call(
    _mid_body,
    grid=(GRID,),
    in_specs=[
        pl.BlockSpec((NC, RBLK, DH), lambda i: (0, i, 0)),
        pl.BlockSpec((NC, RBLK, DH), lambda i: (0, i, 0)),
        pl.BlockSpec((RBLK, D), lambda i: (i, 0)),
        pl.BlockSpec((RBLK, 1), lambda i: (i, 0)),
        pl.BlockSpec((1, D), lambda i: (0, 0)),
        pl.BlockSpec((D, D), lambda i: (0, 0)),
        pl.BlockSpec((D, 1), lambda i: (0, 0)),
    ],
    out_specs=pl.BlockSpec((RBLK, 1), lambda i: (i, 0)),
    out_shape=jax.ShapeDtypeStruct((NPAD, 1), jnp.float32),
)


def _fin_body(accs_ref, sd_ref, dinv_ref, batch_ref, b2_ref, wlin_ref, blin_ref,
              out_ref):
    z = (accs_ref[0] + accs_ref[1] + sd_ref[...]) * dinv_ref[...]
    c2 = jnp.sum(b2_ref[...] * wlin_ref[...])
    zc = z + c2
    ids = batch_ref[...]
    g = lax.broadcasted_iota(jnp.int32, (NG, 1, 1), 0)
    m = ids[None] == g
    sums = jnp.sum(jnp.where(m, zc[None], 0.0), axis=(1, 2))
    out_ref[...] = sums[:, None] + blin_ref[...]


_fin_call = pl.pallas_call(
    _fin_body,
    out_shape=jax.ShapeDtypeStruct((NG, 1), jnp.float32),
)


def kernel(x, edge_index, batch, W1, b1, W2, b2, W_lin, b_lin):
    src = edge_index[0].astype(jnp.int32)
    dst = edge_index[1].astype(jnp.int32)
    pe = EPAD - E
    spread = N + jax.lax.rem(jnp.arange(pe, dtype=jnp.int32), jnp.int32(NPAD - N))
    src_p = jnp.concatenate([src, spread]).reshape(NW, K, B)
    dst_p = jnp.concatenate([dst, spread]).reshape(NW, K, B)
    src_t = src_p.reshape(NS, KT, B)
    dst_t = dst_p.reshape(NS, KT, B)
    x_p = jnp.pad(x, ((0, NPAD - N), (0, 0)))
    batch_p = jnp.concatenate(
        [batch.astype(jnp.int32), jnp.full((NPAD - N,), NG, jnp.int32)]
    ).reshape(NPAD // D, D)

    degp = _deg_call(dst_p)
    y, dinv = _pre_call(x_p, W1, degp[:, :, None])
    accp = _msg_call(y[:, :DH], y[:, DH:], src_t, dst_t)
    sdinv = _mid_call(accp[0], accp[1], y, dinv, b1.reshape(1, D), W2, W_lin)
    accs = _seg_call(src_p, dst_p, sdinv.reshape(NPAD))
    out = _fin_call(
        accs.reshape(NC, NPAD // D, D),
        sdinv.reshape(NPAD // D, D),
        dinv.reshape(NPAD // D, D),
        batch_p,
        jnp.broadcast_to(b2.reshape(D, 1), (D, 1)),
        W_lin,
        b_lin.reshape(1, 1),
    )
    return out


# speedup vs baseline: 39.7901x; 1.0623x over previous
import functools

import jax
import jax.numpy as jnp
from jax import lax
from jax.experimental import pallas as pl
from jax.experimental.pallas import tpu as pltpu
from jax.experimental.pallas import tpu_sc as plsc

N = 10000
NPAD = 10240
D = 128
E = 320000
NG = 64
NC = 2
NS = 16
NW = NC * NS
B = 128
K = 80
EPAD = NW * K * B
ROWS_F = NPAD // NS
DUMMY = N + 8
RBLK = 1024
GRID = NPAD // RBLK

_mesh = plsc.VectorSubcoreMesh(core_axis_name="c", subcore_axis_name="s")


def _zero_vec(ref, n):
    zeros16 = jnp.zeros((16,), jnp.float32)

    def body(i, _):
        ref[pl.ds(i * 16, 16)] = zeros16
        return 0

    lax.fori_loop(0, n // 16, body, 0)


def _deg_body(dst_hbm, out_hbm, dstv, vbuf, acc, _):
    cid = lax.axis_index("c")
    sid = lax.axis_index("s")
    wid = sid * NC + cid
    _zero_vec(vbuf, ROWS_F)
    pltpu.sync_copy(vbuf.at[pl.ds(0, ROWS_F)], acc.at[pl.ds(sid * ROWS_F, ROWS_F)])
    pltpu.sync_copy(dst_hbm.at[wid], dstv)
    ones16 = jnp.ones((16,), jnp.float32)
    for i in range(B // 16):
        vbuf[pl.ds(i * 16, 16)] = ones16
    plsc.subcore_barrier()

    def body(j, _):
        pltpu.sync_copy(vbuf.at[pl.ds(0, B)], acc.at[dstv.at[j]], add=True)
        return 0

    lax.fori_loop(0, K, body, 0)
    plsc.subcore_barrier()
    pltpu.sync_copy(acc.at[pl.ds(sid * ROWS_F, ROWS_F)],
                    out_hbm.at[cid, pl.ds(sid * ROWS_F, ROWS_F)])


_deg_call = pl.kernel(
    _deg_body,
    out_type=jax.ShapeDtypeStruct((NC, NPAD), jnp.float32),
    mesh=_mesh,
    scratch_types=[
        pltpu.VMEM((K, B), jnp.int32),
        pltpu.VMEM((ROWS_F,), jnp.float32),
        pltpu.VMEM_SHARED((NPAD,), jnp.float32),
        pltpu.SemaphoreType.DMA,
    ],
)


DH = D // 2
KT = EPAD // (NS * B)
K0 = 80
NB = 4


def _msg_phase(y_hbm, srcv, dstv, ybuf, acc, gsems, ssems, cs, ce):
    nround = (ce - cs) // NB
    for b in range(NB):
        pltpu.async_copy(y_hbm.at[srcv.at[cs + b]], ybuf.at[b], gsems[b])

    def round_body(t, _):
        j0 = cs + t * NB
        for b in range(NB):
            pltpu.make_async_copy(y_hbm.at[srcv.at[j0 + b]], ybuf.at[b],
                                  gsems[b]).wait()
            pltpu.async_copy(ybuf.at[b], acc.at[dstv.at[j0 + b]], ssems[b],
                             add=True)
        for b in range(NB):
            pltpu.make_async_copy(ybuf.at[b], acc.at[dstv.at[j0 + b]],
                                  ssems[b]).wait()

            @pl.when(t < nround - 1)
            def _():
                pltpu.async_copy(y_hbm.at[srcv.at[j0 + NB + b]], ybuf.at[b],
                                 gsems[b])

        return 0

    lax.fori_loop(0, nround, round_body, 0)


def _msg_zero(zb, acc, sid):
    zeros16 = jnp.zeros((16,), jnp.float32)

    def zrow(i, _):
        for k in range(DH // 16):
            zb[i, pl.ds(k * 16, 16)] = zeros16
        return 0

    lax.fori_loop(0, B, zrow, 0)
    for t in range(ROWS_F // B):
        pltpu.sync_copy(zb, acc.at[pl.ds(sid * ROWS_F + t * B, B)])


def _msg_body(y0_hbm, y1_hbm, src_hbm, dst_hbm, out_hbm, srcv, dstv, ybuf, zb,
              acc, gs0, gs1, gs2, gs3, ss0, ss1, ss2, ss3):
    cid = lax.axis_index("c")
    sid = lax.axis_index("s")
    gsems = (gs0, gs1, gs2, gs3)
    ssems = (ss0, ss1, ss2, ss3)
    _msg_zero(zb, acc, sid)
    pltpu.sync_copy(src_hbm.at[sid], srcv)
    pltpu.sync_copy(dst_hbm.at[sid], dstv)
    plsc.subcore_barrier()
    for h, y_hbm in enumerate((y0_hbm, y1_hbm)):
        @pl.when(cid == 0)
        def _():
            _msg_phase(y_hbm, srcv, dstv, ybuf, acc, gsems, ssems, 0, K0)

        @pl.when(cid == 1)
        def _():
            _msg_phase(y_hbm, srcv, dstv, ybuf, acc, gsems, ssems, K0, KT)

        plsc.subcore_barrier()
        for t in range(ROWS_F // B):
            pltpu.sync_copy(acc.at[pl.ds(sid * ROWS_F + t * B, B)],
                            out_hbm.at[h, cid, pl.ds(sid * ROWS_F + t * B, B)])
        if h == 0:
            plsc.subcore_barrier()
            _msg_zero(zb, acc, sid)
            plsc.subcore_barrier()


_msg_call = pl.kernel(
    _msg_body,
    out_type=jax.ShapeDtypeStruct((2, NC, NPAD, DH), jnp.float32),
    mesh=_mesh,
    compiler_params=pltpu.CompilerParams(use_tc_tiling_on_sc=False),
    scratch_types=[
        pltpu.VMEM((KT, B), jnp.int32),
        pltpu.VMEM((KT, B), jnp.int32),
        pltpu.VMEM((NB, B, DH), jnp.float32),
        pltpu.VMEM((B, DH), jnp.float32),
        pltpu.VMEM_SHARED((NPAD, DH), jnp.float32),
        pltpu.SemaphoreType.DMA,
        pltpu.SemaphoreType.DMA,
        pltpu.SemaphoreType.DMA,
        pltpu.SemaphoreType.DMA,
        pltpu.SemaphoreType.DMA,
        pltpu.SemaphoreType.DMA,
        pltpu.SemaphoreType.DMA,
        pltpu.SemaphoreType.DMA,
    ],
)


def _seg_body(src_hbm, dst_hbm, sd_hbm, out_hbm, srcv, dstv, sdv, svals, vbuf, acc, _):
    cid = lax.axis_index("c")
    sid = lax.axis_index("s")
    wid = sid * NC + cid
    _zero_vec(vbuf, ROWS_F)
    pltpu.sync_copy(vbuf.at[pl.ds(0, ROWS_F)], acc.at[pl.ds(sid * ROWS_F, ROWS_F)])
    pltpu.sync_copy(src_hbm.at[wid], srcv)
    pltpu.sync_copy(dst_hbm.at[wid], dstv)
    pltpu.sync_copy(sd_hbm, sdv)
    plsc.subcore_barrier()

    def body(j, _):
        for i in range(B // 16):
            idx = srcv[j, pl.ds(i * 16, 16)]
            hi = lax.shift_right_logical(idx, 7)
            lo = lax.bitwise_and(idx, 127)
            svals[pl.ds(i * 16, 16)] = plsc.load_gather(sdv, [hi, lo])
        pltpu.sync_copy(svals, acc.at[dstv.at[j]], add=True)
        return 0

    lax.fori_loop(0, K, body, 0)
    plsc.subcore_barrier()
    pltpu.sync_copy(acc.at[pl.ds(sid * ROWS_F, ROWS_F)],
                    out_hbm.at[cid, pl.ds(sid * ROWS_F, ROWS_F)])


_seg_call = pl.kernel(
    _seg_body,
    out_type=jax.ShapeDtypeStruct((NC, NPAD), jnp.float32),
    mesh=_mesh,
    compiler_params=pltpu.CompilerParams(needs_layout_passes=False),
    scratch_types=[
        pltpu.VMEM((K, B), jnp.int32),
        pltpu.VMEM((K, B), jnp.int32),
        pltpu.VMEM((NPAD // D, D), jnp.float32),
        pltpu.VMEM((B,), jnp.float32),
        pltpu.VMEM((ROWS_F,), jnp.float32),
        pltpu.VMEM_SHARED((NPAD,), jnp.float32),
        pltpu.SemaphoreType.DMA,
    ],
)


def _pre_body(x_ref, w1_ref, degp_ref, y0_ref, y1_ref, dinv_ref, dinv2_ref):
    dp = degp_ref[...]
    deg_row = dp[0:1, :] + dp[1:2, :] + 1.0
    dinv_row = lax.rsqrt(deg_row)
    dinv = jnp.transpose(dinv_row, (1, 0))
    xw = jnp.dot(x_ref[...], w1_ref[...], preferred_element_type=jnp.float32)
    y = xw * dinv
    y0_ref[...] = y[:, :DH]
    y1_ref[...] = y[:, DH:]
    dinv_ref[...] = dinv
    dinv2_ref[...] = jnp.reshape(dinv_row, (RBLK // D, D))


_pre_call = pl.pallas_call(
    _pre_body,
    grid=(GRID,),
    in_specs=[
        pl.BlockSpec((RBLK, D), lambda i: (i, 0)),
        pl.BlockSpec((D, D), lambda i: (0, 0)),
        pl.BlockSpec((NC, RBLK), lambda i: (0, i)),
    ],
    out_specs=[
        pl.BlockSpec((RBLK, DH), lambda i: (i, 0)),
        pl.BlockSpec((RBLK, DH), lambda i: (i, 0)),
        pl.BlockSpec((RBLK, 1), lambda i: (i, 0)),
        pl.BlockSpec((RBLK // D, D), lambda i: (i, 0)),
    ],
    out_shape=[
        jax.ShapeDtypeStruct((NPAD, DH), jnp.float32),
        jax.ShapeDtypeStruct((NPAD, DH), jnp.float32),
        jax.ShapeDtypeStruct((NPAD, 1), jnp.float32),
        jax.ShapeDtypeStruct((NPAD // D, D), jnp.float32),
    ],
)


def _mid_body(acc0_ref, acc1_ref, y0_ref, y1_ref, dinv_ref, b1_ref, w2_ref,
              wlin_ref, s_ref):
    dinv = dinv_ref[...]
    b1 = b1_ref[...]
    w2l = jnp.dot(w2_ref[...], wlin_ref[...], preferred_element_type=jnp.float32)
    a0 = acc0_ref[0] + acc0_ref[1] + y0_ref[...]
    a1 = acc1_ref[0] + acc1_ref[1] + y1_ref[...]
    h0 = jnp.maximum(a0 * dinv + b1[:, :DH], 0.0)
    h1 = jnp.maximum(a1 * dinv + b1[:, DH:], 0.0)
    s = (jnp.dot(h0, w2l[:DH], preferred_element_type=jnp.float32)
         + jnp.dot(h1, w2l[DH:], preferred_element_type=jnp.float32))
    s_ref[...] = jnp.reshape(s * dinv, (RBLK // D, D))


_mid_call = pl.pallas_call(
    _mid_body,
    grid=(GRID,),
    in_specs=[
        pl.BlockSpec((NC, RBLK, DH), lambda i: (0, i, 0)),
        pl.BlockSpec((NC, RBLK, DH), lambda i: (0, i, 0)),
        pl.BlockSpec((RBLK, DH), lambda i: (i, 0)),
        pl.BlockSpec((RBLK, DH), lambda i: (i, 0)),
        pl.BlockSpec((RBLK, 1), lambda i: (i, 0)),
        pl.BlockSpec((1, D), lambda i: (0, 0)),
        pl.BlockSpec((D, D), lambda i: (0, 0)),
        pl.BlockSpec((D, 1), lambda i: (0, 0)),
    ],
    out_specs=pl.BlockSpec((RBLK // D, D), lambda i: (i, 0)),
    out_shape=jax.ShapeDtypeStruct((NPAD // D, D), jnp.float32),
)


def _fin_body(accs_ref, sd_ref, dinv_ref, batch_ref, b2_ref, wlin_ref, blin_ref,
              out_ref):
    z = (accs_ref[0] + accs_ref[1] + sd_ref[...]) * dinv_ref[...]
    c2 = jnp.sum(b2_ref[...] * wlin_ref[...])
    zc = z + c2
    ids = batch_ref[...]
    g = lax.broadcasted_iota(jnp.int32, (NG, 1, 1), 0)
    m = ids[None] == g
    sums = jnp.sum(jnp.where(m, zc[None], 0.0), axis=(1, 2))
    out_ref[...] = sums[:, None] + blin_ref[...]


_fin_call = pl.pallas_call(
    _fin_body,
    out_shape=jax.ShapeDtypeStruct((NG, 1), jnp.float32),
)


def kernel(x, edge_index, batch, W1, b1, W2, b2, W_lin, b_lin):
    src = edge_index[0].astype(jnp.int32)
    dst = edge_index[1].astype(jnp.int32)
    pe = EPAD - E
    spread = N + jax.lax.rem(jnp.arange(pe, dtype=jnp.int32), jnp.int32(NPAD - N))
    src_p = jnp.concatenate([src, spread]).reshape(NW, K, B)
    dst_p = jnp.concatenate([dst, spread]).reshape(NW, K, B)
    src_t = src_p.reshape(NS, KT, B)
    dst_t = dst_p.reshape(NS, KT, B)
    x_p = jnp.pad(x, ((0, NPAD - N), (0, 0)))
    batch_p = jnp.concatenate(
        [batch.astype(jnp.int32), jnp.full((NPAD - N,), NG, jnp.int32)]
    ).reshape(NPAD // D, D)

    degp = _deg_call(dst_p)
    y0, y1, dinv, dinv2 = _pre_call(x_p, W1, degp)
    accp = _msg_call(y0, y1, src_t, dst_t)
    sd2 = _mid_call(accp[0], accp[1], y0, y1, dinv, b1.reshape(1, D), W2,
                    W_lin)
    accs = _seg_call(src_p, dst_p, sd2)
    out = _fin_call(
        accs.reshape(NC, NPAD // D, D),
        sd2,
        dinv2,
        batch_p,
        b2.reshape(D, 1),
        W_lin,
        b_lin.reshape(1, 1),
    )
    return out


# speedup vs baseline: 40.0364x; 1.0062x over previous
import functools

import jax
import jax.numpy as jnp
from jax import lax
from jax.experimental import pallas as pl
from jax.experimental.pallas import tpu as pltpu
from jax.experimental.pallas import tpu_sc as plsc

N = 10000
NPAD = 10240
D = 128
E = 320000
NG = 64
NC = 2
NS = 16
NW = NC * NS
B = 128
K = 80
EPAD = NW * K * B
ROWS_F = NPAD // NS
DUMMY = N + 8
RBLK = 1024
GRID = NPAD // RBLK

_mesh = plsc.VectorSubcoreMesh(core_axis_name="c", subcore_axis_name="s")


def _zero_vec(ref, n):
    zeros16 = jnp.zeros((16,), jnp.float32)

    def body(i, _):
        ref[pl.ds(i * 16, 16)] = zeros16
        return 0

    lax.fori_loop(0, n // 16, body, 0)


def _deg_body(dst_hbm, out_hbm, dstv, vbuf, acc, _):
    cid = lax.axis_index("c")
    sid = lax.axis_index("s")
    wid = sid * NC + cid
    _zero_vec(vbuf, ROWS_F)
    pltpu.sync_copy(vbuf.at[pl.ds(0, ROWS_F)], acc.at[pl.ds(sid * ROWS_F, ROWS_F)])
    pltpu.sync_copy(dst_hbm.at[wid], dstv)
    ones16 = jnp.ones((16,), jnp.float32)
    for i in range(B // 16):
        vbuf[pl.ds(i * 16, 16)] = ones16
    plsc.subcore_barrier()

    def body(j, _):
        pltpu.sync_copy(vbuf.at[pl.ds(0, B)], acc.at[dstv.at[j]], add=True)
        return 0

    lax.fori_loop(0, K, body, 0)
    plsc.subcore_barrier()
    pltpu.sync_copy(acc.at[pl.ds(sid * ROWS_F, ROWS_F)],
                    out_hbm.at[cid, pl.ds(sid * ROWS_F, ROWS_F)])


_deg_call = pl.kernel(
    _deg_body,
    out_type=jax.ShapeDtypeStruct((NC, NPAD), jnp.float32),
    mesh=_mesh,
    scratch_types=[
        pltpu.VMEM((K, B), jnp.int32),
        pltpu.VMEM((ROWS_F,), jnp.float32),
        pltpu.VMEM_SHARED((NPAD,), jnp.float32),
        pltpu.SemaphoreType.DMA,
    ],
)


DH = D // 2
KT = EPAD // (NS * B)
K0 = 80
NB = 4


def _msg_phase(y_hbm, srcv, dstv, ybuf, acc, gsems, ssems, cs, ce):
    nround = (ce - cs) // NB
    for b in range(NB):
        pltpu.async_copy(y_hbm.at[srcv.at[cs + b]], ybuf.at[b], gsems[b])

    def round_body(t, _):
        j0 = cs + t * NB
        for b in range(NB):
            pltpu.make_async_copy(y_hbm.at[srcv.at[j0 + b]], ybuf.at[b],
                                  gsems[b]).wait()
            pltpu.async_copy(ybuf.at[b], acc.at[dstv.at[j0 + b]], ssems[b],
                             add=True)
        for b in range(NB):
            pltpu.make_async_copy(ybuf.at[b], acc.at[dstv.at[j0 + b]],
                                  ssems[b]).wait()

            @pl.when(t < nround - 1)
            def _():
                pltpu.async_copy(y_hbm.at[srcv.at[j0 + NB + b]], ybuf.at[b],
                                 gsems[b])

        return 0

    lax.fori_loop(0, nround, round_body, 0)


def _msg_zero(zb, acc, sid):
    zeros16 = jnp.zeros((16,), jnp.float32)

    def zrow(i, _):
        for k in range(DH // 16):
            zb[i, pl.ds(k * 16, 16)] = zeros16
        return 0

    lax.fori_loop(0, B, zrow, 0)
    for t in range(ROWS_F // B):
        pltpu.sync_copy(zb, acc.at[pl.ds(sid * ROWS_F + t * B, B)])


def _msg_body(y0_hbm, y1_hbm, src_hbm, dst_hbm, out_hbm, srcv, dstv, ybuf, zb,
              acc, gs0, gs1, gs2, gs3, ss0, ss1, ss2, ss3):
    cid = lax.axis_index("c")
    sid = lax.axis_index("s")
    gsems = (gs0, gs1, gs2, gs3)
    ssems = (ss0, ss1, ss2, ss3)
    _msg_zero(zb, acc, sid)
    pltpu.sync_copy(src_hbm.at[sid], srcv)
    pltpu.sync_copy(dst_hbm.at[sid], dstv)
    plsc.subcore_barrier()
    for h, y_hbm in enumerate((y0_hbm, y1_hbm)):
        @pl.when(cid == 0)
        def _():
            _msg_phase(y_hbm, srcv, dstv, ybuf, acc, gsems, ssems, 0, K0)

        @pl.when(cid == 1)
        def _():
            _msg_phase(y_hbm, srcv, dstv, ybuf, acc, gsems, ssems, K0, KT)

        plsc.subcore_barrier()
        for t in range(ROWS_F // B):
            pltpu.sync_copy(acc.at[pl.ds(sid * ROWS_F + t * B, B)],
                            out_hbm.at[h, cid, pl.ds(sid * ROWS_F + t * B, B)])
        if h == 0:
            plsc.subcore_barrier()
            _msg_zero(zb, acc, sid)
            plsc.subcore_barrier()


_msg_call = pl.kernel(
    _msg_body,
    out_type=jax.ShapeDtypeStruct((2, NC, NPAD, DH), jnp.float32),
    mesh=_mesh,
    compiler_params=pltpu.CompilerParams(use_tc_tiling_on_sc=False),
    scratch_types=[
        pltpu.VMEM((KT, B), jnp.int32),
        pltpu.VMEM((KT, B), jnp.int32),
        pltpu.VMEM((NB, B, DH), jnp.float32),
        pltpu.VMEM((B, DH), jnp.float32),
        pltpu.VMEM_SHARED((NPAD, DH), jnp.float32),
        pltpu.SemaphoreType.DMA,
        pltpu.SemaphoreType.DMA,
        pltpu.SemaphoreType.DMA,
        pltpu.SemaphoreType.DMA,
        pltpu.SemaphoreType.DMA,
        pltpu.SemaphoreType.DMA,
        pltpu.SemaphoreType.DMA,
        pltpu.SemaphoreType.DMA,
    ],
)


CW = NG + 1
CSZ = CW * NPAD
FW = CSZ // NS
ZB = 4160


def _seg_body(src_hbm, dst_hbm, dinv_hbm, batch_hbm, out_hbm,
              srcv, dstv, dv, bv, fidx, valsb, zbuf, acc, _):
    cid = lax.axis_index("c")
    sid = lax.axis_index("s")
    wid = sid * NC + cid
    _zero_vec(zbuf, ZB)
    for t in range(FW // ZB):
        pltpu.sync_copy(zbuf, acc.at[pl.ds(sid * FW + t * ZB, ZB)])
    pltpu.sync_copy(src_hbm.at[wid], srcv)
    pltpu.sync_copy(dst_hbm.at[wid], dstv)
    pltpu.sync_copy(dinv_hbm, dv)
    pltpu.sync_copy(batch_hbm, bv)
    plsc.subcore_barrier()

    def body(j, _):
        for i in range(B // 16):
            s16 = srcv[j, pl.ds(i * 16, 16)]
            d16 = dstv[j, pl.ds(i * 16, 16)]
            hi = lax.shift_right_logical(d16, 7)
            lo = lax.bitwise_and(d16, 127)
            g16 = plsc.load_gather(bv, [hi, lo])
            v16 = plsc.load_gather(dv, [hi, lo])
            fidx[0, pl.ds(i * 16, 16)] = g16 * NPAD + s16
            valsb[pl.ds(i * 16, 16)] = v16
        pltpu.sync_copy(valsb, acc.at[fidx.at[0]], add=True)
        return 0

    lax.fori_loop(0, K, body, 0)

    iota16 = lax.broadcasted_iota(jnp.int32, (16,), 0)
    nrows = NPAD // NW
    for k in range(3):
        for i in range(8):
            off = k * B + i * 16
            n16 = wid * nrows + off + iota16
            nc16 = jnp.minimum(n16, NPAD - 1)
            hi = lax.shift_right_logical(nc16, 7)
            lo = lax.bitwise_and(nc16, 127)
            g16 = plsc.load_gather(bv, [hi, lo])
            v16 = plsc.load_gather(dv, [hi, lo])
            if off < nrows:
                fidx[0, pl.ds(i * 16, 16)] = g16 * NPAD + n16
            else:
                fidx[0, pl.ds(i * 16, 16)] = NG * NPAD + nc16
            valsb[pl.ds(i * 16, 16)] = v16
        pltpu.sync_copy(valsb, acc.at[fidx.at[0]], add=True)

    plsc.subcore_barrier()
    pltpu.sync_copy(acc.at[pl.ds(sid * FW, FW)],
                    out_hbm.at[cid, pl.ds(sid * FW, FW)])


_seg_call = pl.kernel(
    _seg_body,
    out_type=jax.ShapeDtypeStruct((NC, CSZ), jnp.float32),
    mesh=_mesh,
    compiler_params=pltpu.CompilerParams(needs_layout_passes=False),
    scratch_types=[
        pltpu.VMEM((K, B), jnp.int32),
        pltpu.VMEM((K, B), jnp.int32),
        pltpu.VMEM((NPAD // D, D), jnp.float32),
        pltpu.VMEM((NPAD // D, D), jnp.int32),
        pltpu.VMEM((1, B), jnp.int32),
        pltpu.VMEM((B,), jnp.float32),
        pltpu.VMEM((ZB,), jnp.float32),
        pltpu.VMEM_SHARED((CSZ,), jnp.float32),
        pltpu.SemaphoreType.DMA,
    ],
)


def _pre_body(x_ref, w1_ref, degp_ref, y0_ref, y1_ref, dinv_ref, dinv2_ref):
    dp = degp_ref[...]
    deg_row = dp[0:1, :] + dp[1:2, :] + 1.0
    dinv_row = 1.0 / jnp.sqrt(deg_row)
    dinv = jnp.transpose(dinv_row, (1, 0))
    xw = jnp.dot(x_ref[...], w1_ref[...], preferred_element_type=jnp.float32)
    y = xw * dinv
    y0_ref[...] = y[:, :DH]
    y1_ref[...] = y[:, DH:]
    dinv_ref[...] = dinv
    dinv2_ref[...] = jnp.reshape(dinv_row, (RBLK // D, D))


_pre_call = pl.pallas_call(
    _pre_body,
    grid=(GRID,),
    in_specs=[
        pl.BlockSpec((RBLK, D), lambda i: (i, 0)),
        pl.BlockSpec((D, D), lambda i: (0, 0)),
        pl.BlockSpec((NC, RBLK), lambda i: (0, i)),
    ],
    out_specs=[
        pl.BlockSpec((RBLK, DH), lambda i: (i, 0)),
        pl.BlockSpec((RBLK, DH), lambda i: (i, 0)),
        pl.BlockSpec((RBLK, 1), lambda i: (i, 0)),
        pl.BlockSpec((RBLK // D, D), lambda i: (i, 0)),
    ],
    out_shape=[
        jax.ShapeDtypeStruct((NPAD, DH), jnp.float32),
        jax.ShapeDtypeStruct((NPAD, DH), jnp.float32),
        jax.ShapeDtypeStruct((NPAD, 1), jnp.float32),
        jax.ShapeDtypeStruct((NPAD // D, D), jnp.float32),
    ],
)


def _mid_body(acc0_ref, acc1_ref, y0_ref, y1_ref, dinv_ref, b1_ref, w2_ref,
              ud_ref):
    dinv = dinv_ref[...]
    b1 = b1_ref[...]
    a0 = acc0_ref[0] + acc0_ref[1] + y0_ref[...]
    a1 = acc1_ref[0] + acc1_ref[1] + y1_ref[...]
    h0 = jnp.maximum(a0 * dinv + b1[:, :DH], 0.0)
    h1 = jnp.maximum(a1 * dinv + b1[:, DH:], 0.0)
    u = (jnp.dot(h0, w2_ref[:DH, :], preferred_element_type=jnp.float32)
         + jnp.dot(h1, w2_ref[DH:, :], preferred_element_type=jnp.float32))
    ud_ref[...] = u * dinv


_mid_call = pl.pallas_call(
    _mid_body,
    grid=(GRID,),
    in_specs=[
        pl.BlockSpec((NC, RBLK, DH), lambda i: (0, i, 0)),
        pl.BlockSpec((NC, RBLK, DH), lambda i: (0, i, 0)),
        pl.BlockSpec((RBLK, DH), lambda i: (i, 0)),
        pl.BlockSpec((RBLK, DH), lambda i: (i, 0)),
        pl.BlockSpec((RBLK, 1), lambda i: (i, 0)),
        pl.BlockSpec((1, D), lambda i: (0, 0)),
        pl.BlockSpec((D, D), lambda i: (0, 0)),
    ],
    out_specs=pl.BlockSpec((RBLK, D), lambda i: (i, 0)),
    out_shape=jax.ShapeDtypeStruct((NPAD, D), jnp.float32),
)


def _fin_body(cp_ref, ud_ref, batch_ref, b2_ref, wlin_ref, blin_ref, out_ref):
    c = cp_ref[0] + cp_ref[1]
    pooled_a = jnp.dot(c, ud_ref[...], preferred_element_type=jnp.float32,
                       precision=lax.Precision.HIGHEST)
    ids = batch_ref[...]
    g = lax.broadcasted_iota(jnp.int32, (NG, 1, 1), 0)
    ng = jnp.sum(jnp.where(ids[None] == g, 1.0, 0.0), axis=(1, 2))
    pooled = pooled_a + ng[:, None] * b2_ref[...]
    out_ref[...] = (jnp.dot(pooled, wlin_ref[...],
                            preferred_element_type=jnp.float32)
                    + blin_ref[...])


_fin_call = pl.pallas_call(
    _fin_body,
    grid=(1,),
    in_specs=[
        pl.BlockSpec((NC, NG, NPAD), lambda i: (0, 0, 0)),
        pl.BlockSpec((NPAD, D), lambda i: (0, 0)),
        pl.BlockSpec((NPAD // D, D), lambda i: (0, 0)),
        pl.BlockSpec((1, D), lambda i: (0, 0)),
        pl.BlockSpec((D, 1), lambda i: (0, 0)),
        pl.BlockSpec((1, 1), lambda i: (0, 0)),
    ],
    out_specs=pl.BlockSpec((NG, 1), lambda i: (0, 0)),
    out_shape=jax.ShapeDtypeStruct((NG, 1), jnp.float32),
)


def kernel(x, edge_index, batch, W1, b1, W2, b2, W_lin, b_lin):
    src = edge_index[0].astype(jnp.int32)
    dst = edge_index[1].astype(jnp.int32)
    pe = EPAD - E
    spread = N + jax.lax.rem(jnp.arange(pe, dtype=jnp.int32), jnp.int32(NPAD - N))
    src_p = jnp.concatenate([src, spread]).reshape(NW, K, B)
    dst_p = jnp.concatenate([dst, spread]).reshape(NW, K, B)
    src_t = src_p.reshape(NS, KT, B)
    dst_t = dst_p.reshape(NS, KT, B)
    x_p = jnp.pad(x, ((0, NPAD - N), (0, 0)))
    batch_p = jnp.concatenate(
        [batch.astype(jnp.int32), jnp.full((NPAD - N,), NG, jnp.int32)]
    ).reshape(NPAD // D, D)

    degp = _deg_call(dst_p)
    y0, y1, dinv, dinv2 = _pre_call(x_p, W1, degp)
    accp = _msg_call(y0, y1, src_t, dst_t)
    udinv = _mid_call(accp[0], accp[1], y0, y1, dinv, b1.reshape(1, D), W2)
    cpart = _seg_call(src_p, dst_p, dinv2, batch_p)
    out = _fin_call(
        cpart.reshape(NC, CW, NPAD),
        udinv,
        batch_p,
        b2.reshape(1, D),
        W_lin,
        b_lin.reshape(1, 1),
    )
    return out
